# baseline (speedup 1.0000x reference)
import jax
import jax.numpy as jnp
from jax import lax
from jax.experimental import pallas as pl
from jax.experimental.pallas import tpu as pltpu
from jax.experimental.pallas import tpu_sc as plsc

CHUNKS = 8
RES_DIM = 2048
BIAS = 1.6
C1, C3, C5 = 1.0, -1.0 / 3.0, 2.0 / 15.0

SC_CHUNKS = 2
TC_CHUNKS = CHUNKS - SC_CHUNKS

NC, NS, L = 2, 16, 16
NW = NC * NS
SC_ROWS = SC_CHUNKS * RES_DIM
RPW = SC_ROWS // NW
GROUP = 16
NGRP = RPW // GROUP
JBLK = RES_DIM // L

TI = 1024


def _poly(pre):
    p2 = pre * pre
    return pre * (C1 + p2 * (C3 + p2 * C5))


def _sc_body(pv_hbm, s_hbm, wr_hbm, out_hbm,
             wrbuf, sbuf, pvbuf, obuf, tbuf, sem0, sem1):
    wid = lax.axis_index("s") * NC + lax.axis_index("c")
    row0 = wid * RPW
    chunk = row0 // RES_DIM

    pltpu.sync_copy(s_hbm.at[pl.ds(chunk * RES_DIM, RES_DIM)], sbuf)
    pltpu.sync_copy(pv_hbm.at[pl.ds(row0, RPW)], pvbuf)

    sems = (sem0, sem1)
    iota16 = lax.iota(jnp.int32, L)
    zeros = jnp.zeros((L,), jnp.float32)
    tbuf[pl.ds(0, L)] = zeros
    tbuf[pl.ds(2 * L, L)] = zeros

    def shift_dn(v, k):
        tbuf[pl.ds(L, L)] = v
        return tbuf[pl.ds(L + k, L)]

    def shift_up(v, k):
        tbuf[pl.ds(L, L)] = v
        return tbuf[pl.ds(L - k, L)]

    def bitrev4(i):
        return ((i & 1) << 3) | ((i & 2) << 1) | ((i & 4) >> 1) | ((i & 8) >> 3)

    def start(g):
        b = g % 2
        return pltpu.async_copy(
            wr_hbm.at[pl.ds(row0 + g * GROUP, GROUP)], wrbuf.at[b], sems[b])

    handles = {0: start(0)}

    for g in range(NGRP):
        b = g % 2
        handles.pop(g).wait()
        if g + 1 < NGRP:
            handles[g + 1] = start(g + 1)

        def jb_body(jb, accs):
            base = jb * L
            sv = sbuf[pl.ds(base, L)]
            return tuple(accs[r] + wrbuf[b, r, pl.ds(base, L)] * sv
                         for r in range(GROUP))

        accs = lax.fori_loop(
            0, JBLK, jb_body,
            tuple(jnp.zeros((L,), jnp.float32) for _ in range(GROUP)))

        vecs = [accs[bitrev4(i)] for i in range(GROUP)]
        k = L // 2
        while len(vecs) > 1:
            keep = (iota16 & k) == 0
            nxt = []
            for i in range(0, len(vecs), 2):
                a = vecs[i] + shift_dn(vecs[i], k)
                b2 = vecs[i + 1] + shift_dn(vecs[i + 1], k)
                nxt.append(jnp.where(keep, a, shift_up(b2, k)))
            vecs = nxt
            k //= 2
        dots = vecs[0]

        pre = dots + pvbuf[pl.ds(g * GROUP, L)] + BIAS
        obuf[pl.ds(g * GROUP, L)] = _poly(pre)

    pltpu.sync_copy(obuf, out_hbm.at[pl.ds(row0, RPW)])


def _sc_matvec(pv_flat, s_flat, wr_flat):
    mesh = plsc.VectorSubcoreMesh(core_axis_name="c", subcore_axis_name="s")
    sc_call = pl.kernel(
        _sc_body,
        mesh=mesh,
        out_type=jax.ShapeDtypeStruct((SC_ROWS,), jnp.float32),
        scratch_types=[
            pltpu.VMEM((2, GROUP, RES_DIM), jnp.float32),
            pltpu.VMEM((RES_DIM,), jnp.float32),
            pltpu.VMEM((RPW,), jnp.float32),
            pltpu.VMEM((RPW,), jnp.float32),
            pltpu.VMEM((GROUP * L,), jnp.float32),
            pltpu.SemaphoreType.DMA,
            pltpu.SemaphoreType.DMA,
        ],
    )
    return sc_call(pv_flat, s_flat, wr_flat)


def _tc_body(pv_ref, s_ref, wr_ref, out_ref):
    w = wr_ref[0]
    s = s_ref[0]
    pre = jnp.sum(w * s, axis=1)
    pre = pre + pv_ref[0, 0, 0] + BIAS
    out_ref[0, 0, 0] = _poly(pre)


def _tc_matvec(pv, s, wr):
    n_chunks = wr.shape[0]
    nb = RES_DIM // TI
    s2 = s[:, None, :]
    pv4 = pv.reshape(n_chunks, nb, 1, TI)
    out = pl.pallas_call(
        _tc_body,
        grid=(n_chunks, nb),
        in_specs=[
            pl.BlockSpec((1, 1, 1, TI), lambda c, i: (c, i, 0, 0)),
            pl.BlockSpec((1, 1, RES_DIM), lambda c, i: (c, 0, 0)),
            pl.BlockSpec((1, TI, RES_DIM), lambda c, i: (c, i, 0)),
        ],
        out_specs=pl.BlockSpec((1, 1, 1, TI), lambda c, i: (c, i, 0, 0)),
        out_shape=jax.ShapeDtypeStruct((n_chunks, nb, 1, TI), jnp.float32),
    )(pv4, s2, wr)
    return out.reshape(n_chunks, RES_DIM)


def kernel(proj_vars, res_state, wr):
    out_sc = _sc_matvec(
        proj_vars[:SC_CHUNKS].reshape(-1),
        res_state[:SC_CHUNKS].reshape(-1),
        wr[:SC_CHUNKS].reshape(SC_ROWS, RES_DIM),
    ).reshape(SC_CHUNKS, RES_DIM)
    out_tc = _tc_matvec(
        proj_vars[SC_CHUNKS:], res_state[SC_CHUNKS:], wr[SC_CHUNKS:])
    return jnp.concatenate([out_sc, out_tc], axis=0)


# speedup vs baseline: 3.3304x; 3.3304x over previous
import jax
import jax.numpy as jnp
from jax.experimental import pallas as pl
from jax.experimental.pallas import tpu as pltpu

CHUNKS = 8
RES_DIM = 2048
BIAS = 1.6
C1, C3, C5 = 1.0, -1.0 / 3.0, 2.0 / 15.0

NBLK = 64
BROWS = (CHUNKS * RES_DIM) // NBLK
RING = 8


def _poly(pre):
    p2 = pre * pre
    return pre * (C1 + p2 * (C3 + p2 * C5))


def _ring_body(pv_ref, s_ref, wr_hbm, out_ref, buf, sems, sems2):
    half = BROWS // 2

    def _copies(b):
        sl = b % RING
        return (
            pltpu.make_async_copy(
                wr_hbm.at[pl.ds(b * BROWS, half), :],
                buf.at[sl, pl.ds(0, half)],
                sems.at[sl],
            ),
            pltpu.make_async_copy(
                wr_hbm.at[pl.ds(b * BROWS + half, half), :],
                buf.at[sl, pl.ds(half, half)],
                sems2.at[sl],
            ),
        )

    def start(b):
        for cp in _copies(b):
            cp.start()

    for b in range(RING - 1):
        start(b)

    for b in range(NBLK):
        for cp in _copies(b):
            cp.wait()
        if b + RING - 1 < NBLK:
            start(b + RING - 1)
        c = (b * BROWS) // RES_DIM
        w = buf[b % RING]
        s_row = s_ref[pl.ds(c, 1), :]
        pre = jnp.sum(w * s_row, axis=1) + pv_ref[b, 0] + BIAS
        out_ref[b, 0] = _poly(pre)


def kernel(proj_vars, res_state, wr):
    out = pl.pallas_call(
        _ring_body,
        in_specs=[
            pl.BlockSpec(memory_space=pltpu.VMEM),
            pl.BlockSpec(memory_space=pltpu.VMEM),
            pl.BlockSpec(memory_space=pl.ANY),
        ],
        out_specs=pl.BlockSpec(memory_space=pltpu.VMEM),
        out_shape=jax.ShapeDtypeStruct((NBLK, 1, BROWS), jnp.float32),
        scratch_shapes=[
            pltpu.VMEM((RING, BROWS, RES_DIM), jnp.float32),
            pltpu.SemaphoreType.DMA((RING,)),
            pltpu.SemaphoreType.DMA((RING,)),
        ],
    )(proj_vars.reshape(NBLK, 1, BROWS), res_state,
      wr.reshape(CHUNKS * RES_DIM, RES_DIM))
    return out.reshape(CHUNKS, RES_DIM)


# speedup vs baseline: 3.3356x; 1.0016x over previous
import jax
import jax.numpy as jnp
from jax.experimental import pallas as pl
from jax.experimental.pallas import tpu as pltpu

CHUNKS = 8
RES_DIM = 2048
BIAS = 1.6
C1, C3, C5 = 1.0, -1.0 / 3.0, 2.0 / 15.0

NBLK = 64
BROWS = (CHUNKS * RES_DIM) // NBLK
RING = 8


def _poly(pre):
    p2 = pre * pre
    return pre * (C1 + p2 * (C3 + p2 * C5))


def _ring_body(pv_ref, s_ref, wr_hbm, out_ref, buf, sems, sems2):
    half = BROWS // 2

    def _phys(b):
        return (b % 2) * (NBLK // 2) + b // 2

    def _copies(b):
        sl = b % RING
        p = _phys(b)
        return (
            pltpu.make_async_copy(
                wr_hbm.at[pl.ds(p * BROWS, half), :],
                buf.at[sl, pl.ds(0, half)],
                sems.at[sl],
            ),
            pltpu.make_async_copy(
                wr_hbm.at[pl.ds(p * BROWS + half, half), :],
                buf.at[sl, pl.ds(half, half)],
                sems2.at[sl],
            ),
        )

    def start(b):
        for cp in _copies(b):
            cp.start()

    for b in range(RING - 1):
        start(b)

    for b in range(NBLK):
        for cp in _copies(b):
            cp.wait()
        if b + RING - 1 < NBLK:
            start(b + RING - 1)
        p = _phys(b)
        c = (p * BROWS) // RES_DIM
        w = buf[b % RING]
        s_row = s_ref[pl.ds(c, 1), :]
        pre = jnp.sum(w * s_row, axis=1) + pv_ref[p, 0] + BIAS
        out_ref[p, 0] = _poly(pre)


def kernel(proj_vars, res_state, wr):
    out = pl.pallas_call(
        _ring_body,
        in_specs=[
            pl.BlockSpec(memory_space=pltpu.VMEM),
            pl.BlockSpec(memory_space=pltpu.VMEM),
            pl.BlockSpec(memory_space=pl.ANY),
        ],
        out_specs=pl.BlockSpec(memory_space=pltpu.VMEM),
        out_shape=jax.ShapeDtypeStruct((NBLK, 1, BROWS), jnp.float32),
        scratch_shapes=[
            pltpu.VMEM((RING, BROWS, RES_DIM), jnp.float32),
            pltpu.SemaphoreType.DMA((RING,)),
            pltpu.SemaphoreType.DMA((RING,)),
        ],
    )(proj_vars.reshape(NBLK, 1, BROWS), res_state,
      wr.reshape(CHUNKS * RES_DIM, RES_DIM))
    return out.reshape(CHUNKS, RES_DIM)
